# baseline (speedup 1.0000x reference)
import jax
import jax.numpy as jnp
from jax import lax
from jax.experimental import pallas as pl

D_MODEL = 768
NUM_EXPERTS = 8
TOP_K = 2

_RSPLIT = 2
_CSPLIT = 6
_SUBR = 2048
_BR = _SUBR * _RSPLIT
_DC = D_MODEL // _CSPLIT


def _route_t(acc):
    e_idx = lax.broadcasted_iota(jnp.int32, acc.shape, 0)
    m1 = jnp.max(acc, axis=0, keepdims=True)
    i1 = jnp.min(jnp.where(acc == m1, e_idx, NUM_EXPERTS),
                 axis=0, keepdims=True)
    neg = jnp.float32(-jnp.inf)
    masked = jnp.where(e_idx == i1, neg, acc)
    m2 = jnp.max(masked, axis=0, keepdims=True)
    i2 = jnp.min(jnp.where(masked == m2, e_idx, NUM_EXPERTS),
                 axis=0, keepdims=True)
    t = jnp.exp(m2 - m1)
    denom = 1.0 + t
    return (jnp.concatenate([1.0 / denom, t / denom], axis=0),
            jnp.concatenate([i1, i2], axis=0))


def _router_body(*refs):
    x_refs = refs[:_RSPLIT * _CSPLIT]
    w_ref = refs[_RSPLIT * _CSPLIT]
    probs_ref, idx_ref = refs[_RSPLIT * _CSPLIT + 1:]

    ps, idxs = [], []
    for r in range(_RSPLIT):
        acc = None
        for c in range(_CSPLIT):
            part = lax.dot_general(
                w_ref[:, pl.ds(c * _DC, _DC)], x_refs[r * _CSPLIT + c][...],
                (((1,), (1,)), ((), ())),
                preferred_element_type=jnp.float32)
            acc = part if acc is None else acc + part
        p, i = _route_t(acc)
        ps.append(p)
        idxs.append(i)
    probs_ref[...] = jnp.concatenate(ps, axis=1)
    idx_ref[...] = jnp.concatenate(idxs, axis=1)


def kernel(x, W):
    b, s, d = x.shape
    n = b * s
    x_flat = x.reshape(n, d)

    def x_spec(r, c):
        return pl.BlockSpec(
            (_SUBR, _DC), lambda i, r=r, c=c: (i * _RSPLIT + r, c))

    probs_t, idx_t = pl.pallas_call(
        _router_body,
        grid=(n // _BR,),
        in_specs=[x_spec(r, c) for r in range(_RSPLIT)
                  for c in range(_CSPLIT)]
        + [pl.BlockSpec((NUM_EXPERTS, d), lambda i: (0, 0))],
        out_specs=[
            pl.BlockSpec((TOP_K, _BR), lambda i: (0, i)),
            pl.BlockSpec((TOP_K, _BR), lambda i: (0, i)),
        ],
        out_shape=[
            jax.ShapeDtypeStruct((TOP_K, n), jnp.float32),
            jax.ShapeDtypeStruct((TOP_K, n), jnp.int32),
        ],
    )(*([x_flat] * (_RSPLIT * _CSPLIT)), W)
    return probs_t, idx_t


# speedup vs baseline: 1.0003x; 1.0003x over previous
import jax
import jax.numpy as jnp
from jax import lax
from jax.experimental import pallas as pl

D_MODEL = 768
NUM_EXPERTS = 8
TOP_K = 2

_RSPLIT = 4
_CSPLIT = 3
_SUBR = 1024
_BR = _SUBR * _RSPLIT
_DC = D_MODEL // _CSPLIT


def _route_t(acc):
    e_idx = lax.broadcasted_iota(jnp.int32, acc.shape, 0)
    m1 = jnp.max(acc, axis=0, keepdims=True)
    i1 = jnp.min(jnp.where(acc == m1, e_idx, NUM_EXPERTS),
                 axis=0, keepdims=True)
    neg = jnp.float32(-jnp.inf)
    masked = jnp.where(e_idx == i1, neg, acc)
    m2 = jnp.max(masked, axis=0, keepdims=True)
    i2 = jnp.min(jnp.where(masked == m2, e_idx, NUM_EXPERTS),
                 axis=0, keepdims=True)
    t = jnp.exp(m2 - m1)
    denom = 1.0 + t
    return (jnp.concatenate([1.0 / denom, t / denom], axis=0),
            jnp.concatenate([i1, i2], axis=0))


def _router_body(*refs):
    x_refs = refs[:_RSPLIT * _CSPLIT]
    w_ref = refs[_RSPLIT * _CSPLIT]
    probs_ref, idx_ref = refs[_RSPLIT * _CSPLIT + 1:]

    ps, idxs = [], []
    for r in range(_RSPLIT):
        acc = None
        for c in range(_CSPLIT):
            part = lax.dot_general(
                w_ref[:, pl.ds(c * _DC, _DC)], x_refs[r * _CSPLIT + c][...],
                (((1,), (1,)), ((), ())),
                preferred_element_type=jnp.float32)
            acc = part if acc is None else acc + part
        p, i = _route_t(acc)
        ps.append(p)
        idxs.append(i)
    probs_ref[...] = jnp.concatenate(ps, axis=1)
    idx_ref[...] = jnp.concatenate(idxs, axis=1)


def kernel(x, W):
    b, s, d = x.shape
    n = b * s
    x_flat = x.reshape(n, d)

    def x_spec(r, c):
        return pl.BlockSpec(
            (_SUBR, _DC), lambda i, r=r, c=c: (i * _RSPLIT + r, c))

    probs_t, idx_t = pl.pallas_call(
        _router_body,
        grid=(n // _BR,),
        in_specs=[x_spec(r, c) for r in range(_RSPLIT)
                  for c in range(_CSPLIT)]
        + [pl.BlockSpec((NUM_EXPERTS, d), lambda i: (0, 0))],
        out_specs=[
            pl.BlockSpec((TOP_K, _BR), lambda i: (0, i)),
            pl.BlockSpec((TOP_K, _BR), lambda i: (0, i)),
        ],
        out_shape=[
            jax.ShapeDtypeStruct((TOP_K, n), jnp.float32),
            jax.ShapeDtypeStruct((TOP_K, n), jnp.int32),
        ],
    )(*([x_flat] * (_RSPLIT * _CSPLIT)), W)
    return probs_t.T, idx_t.T


# speedup vs baseline: 1.0167x; 1.0164x over previous
import jax
import jax.numpy as jnp
from jax import lax
from jax.experimental import pallas as pl

D_MODEL = 768
NUM_EXPERTS = 8
TOP_K = 2

_RSPLIT = 8
_CSPLIT = 1
_SUBR = 512
_BR = _SUBR * _RSPLIT
_DC = D_MODEL // _CSPLIT


def _route_t(acc):
    e_idx = lax.broadcasted_iota(jnp.int32, acc.shape, 0)
    m1 = jnp.max(acc, axis=0, keepdims=True)
    i1 = jnp.min(jnp.where(acc == m1, e_idx, NUM_EXPERTS),
                 axis=0, keepdims=True)
    neg = jnp.float32(-jnp.inf)
    masked = jnp.where(e_idx == i1, neg, acc)
    m2 = jnp.max(masked, axis=0, keepdims=True)
    i2 = jnp.min(jnp.where(masked == m2, e_idx, NUM_EXPERTS),
                 axis=0, keepdims=True)
    t = jnp.exp(m2 - m1)
    denom = 1.0 + t
    return (jnp.concatenate([1.0 / denom, t / denom], axis=0),
            jnp.concatenate([i1, i2], axis=0))


def _router_body(*refs):
    x_refs = refs[:_RSPLIT * _CSPLIT]
    w_ref = refs[_RSPLIT * _CSPLIT]
    probs_ref, idx_ref = refs[_RSPLIT * _CSPLIT + 1:]

    ps, idxs = [], []
    for r in range(_RSPLIT):
        acc = None
        for c in range(_CSPLIT):
            part = lax.dot_general(
                w_ref[:, pl.ds(c * _DC, _DC)], x_refs[r * _CSPLIT + c][...],
                (((1,), (1,)), ((), ())),
                preferred_element_type=jnp.float32)
            acc = part if acc is None else acc + part
        p, i = _route_t(acc)
        ps.append(p)
        idxs.append(i)
    probs_ref[...] = jnp.concatenate(ps, axis=1)
    idx_ref[...] = jnp.concatenate(idxs, axis=1)


def kernel(x, W):
    b, s, d = x.shape
    n = b * s
    x_flat = x.reshape(n, d)

    def x_spec(r, c):
        return pl.BlockSpec(
            (_SUBR, _DC), lambda i, r=r, c=c: (i * _RSPLIT + r, c))

    probs_t, idx_t = pl.pallas_call(
        _router_body,
        grid=(n // _BR,),
        in_specs=[x_spec(r, c) for r in range(_RSPLIT)
                  for c in range(_CSPLIT)]
        + [pl.BlockSpec((NUM_EXPERTS, d), lambda i: (0, 0))],
        out_specs=[
            pl.BlockSpec((TOP_K, _BR), lambda i: (0, i)),
            pl.BlockSpec((TOP_K, _BR), lambda i: (0, i)),
        ],
        out_shape=[
            jax.ShapeDtypeStruct((TOP_K, n), jnp.float32),
            jax.ShapeDtypeStruct((TOP_K, n), jnp.int32),
        ],
    )(*([x_flat] * (_RSPLIT * _CSPLIT)), W)
    return probs_t.T, idx_t.T
